# baseline (device time: 433882 ns/iter reference)
import functools

import jax
import jax.numpy as jnp
from jax import lax
from jax.experimental import pallas as pl
from jax.experimental.pallas import tpu as pltpu

N_DEV = 16
M, K, N = 4096, 4096, 8192
K_SH = K // N_DEV
BM, BN = 1024, 2048
N_A = 2048
N_B = N - N_A
SUBS = 4


def _allgather(x, w_mat):

    def body(x_ref, w_ref, xg, wg, xc, wc, xs_sems, xr_sems, ws_sems,
             wr_sems, vs_sems, vr_sems, cp_sem):
        my = lax.axis_index("i")
        left = (my + N_DEV - 1) % N_DEV
        right = (my + 1) % N_DEV

        xc[...] = x_ref[...].astype(jnp.float8_e4m3fn)
        wc[...] = w_ref[...].astype(jnp.float8_e4m3fn)
        cpx = pltpu.make_async_copy(
            xc, xg.at[:, pl.ds(my * K_SH, K_SH)], cp_sem)
        cpx.start()
        cpx.wait()
        cpw = pltpu.make_async_copy(
            wc, wg.at[pl.ds(my * K_SH, K_SH), :], cp_sem)
        cpw.start()
        cpw.wait()

        barrier = pltpu.get_barrier_semaphore()
        for nbr in (left, right):
            pl.semaphore_signal(barrier, inc=1, device_id=(nbr,),
                                device_id_type=pl.DeviceIdType.MESH)
        pl.semaphore_wait(barrier, 2)

        M_H = M // SUBS
        K_H = K_SH // SUBS

        def _desc(stream, h, s, chunk, nbr):
            if stream == 0:
                ref = xg.at[pl.ds(s * M_H, M_H),
                            pl.ds(chunk * K_SH, K_SH)]
                sems = (xs_sems, xr_sems)
            elif stream == 1:
                ref = wg.at[pl.ds(chunk * K_SH + s * K_H, K_H),
                            pl.ds(0, N_A)]
                sems = (ws_sems, wr_sems)
            else:
                ref = wg.at[pl.ds(chunk * K_SH + s * K_H, K_H),
                            pl.ds(N_A, N_B)]
                sems = (vs_sems, vr_sems)
            return pltpu.make_async_remote_copy(
                src_ref=ref, dst_ref=ref,
                send_sem=sems[0].at[h, s], recv_sem=sems[1].at[h, s],
                device_id=(nbr,), device_id_type=pl.DeviceIdType.MESH)

        def _send(stream, h, s):
            chunk = ((my + N_DEV - h) if stream < 2 else (my + h)) % N_DEV
            _desc(stream, h, s, chunk, right if stream < 2 else left).start()

        def _wait_recv(stream, h, s):
            chunk = ((my + 2 * N_DEV - h - 1) if stream < 2
                     else (my + h + 1)) % N_DEV
            _desc(stream, h, s, chunk, left if stream < 2 else right
                  ).wait_recv()

        def _wait_send(stream, h, s):
            chunk = ((my + N_DEV - h) if stream < 2 else (my + h)) % N_DEV
            _desc(stream, h, s, chunk, right if stream < 2 else left
                  ).wait_send()

        for stream in range(3):
            for s in range(SUBS):
                _send(stream, 0, s)

        def hop(h, carry):
            for stream in range(3):
                for s in range(SUBS):
                    _wait_recv(stream, h - 1, s)
                    _send(stream, h, s)
                    _wait_send(stream, h - 1, s)
            return carry

        lax.fori_loop(1, N_DEV - 1, hop, 0)
        for stream in range(3):
            for s in range(SUBS):
                _wait_recv(stream, N_DEV - 2, s)
                _wait_send(stream, N_DEV - 2, s)

        @functools.partial(pl.run_scoped, sem2=pltpu.SemaphoreType.REGULAR)
        def _(sem2):
            for nbr in (left, right):
                pl.semaphore_signal(sem2, inc=1, device_id=(nbr,),
                                    device_id_type=pl.DeviceIdType.MESH)
            pl.semaphore_wait(sem2, 2)

    fp8 = jnp.float8_e4m3fn
    return pl.pallas_call(
        body,
        out_shape=(
            jax.ShapeDtypeStruct((M, K), fp8),
            jax.ShapeDtypeStruct((K, N), fp8),
        ),
        in_specs=[
            pl.BlockSpec(memory_space=pltpu.VMEM),
            pl.BlockSpec(memory_space=pltpu.VMEM),
        ],
        out_specs=(
            pl.BlockSpec(memory_space=pl.ANY),
            pl.BlockSpec(memory_space=pl.ANY),
        ),
        scratch_shapes=[
            pltpu.VMEM((M, K_SH), fp8),
            pltpu.VMEM((K_SH, N), fp8),
            pltpu.SemaphoreType.DMA((N_DEV - 1, SUBS)),
            pltpu.SemaphoreType.DMA((N_DEV - 1, SUBS)),
            pltpu.SemaphoreType.DMA((N_DEV - 1, SUBS)),
            pltpu.SemaphoreType.DMA((N_DEV - 1, SUBS)),
            pltpu.SemaphoreType.DMA((N_DEV - 1, SUBS)),
            pltpu.SemaphoreType.DMA((N_DEV - 1, SUBS)),
            pltpu.SemaphoreType.DMA,
        ],
        compiler_params=pltpu.CompilerParams(collective_id=0),
    )(x, w_mat)


def _gemm(xg, wg, scale):

    def body(s_ref, a_ref, b_ref, o_ref):
        acc = jnp.dot(a_ref[...], b_ref[...],
                      preferred_element_type=jnp.float32)
        o_ref[...] = acc * s_ref[0, 0]

    return pl.pallas_call(
        body,
        grid=(N // BN, M // BM),
        in_specs=[
            pl.BlockSpec(memory_space=pltpu.SMEM),
            pl.BlockSpec((BM, K), lambda j, i: (i, 0)),
            pl.BlockSpec((K, BN), lambda j, i: (0, j)),
        ],
        out_specs=pl.BlockSpec((BM, BN), lambda j, i: (i, j)),
        out_shape=jax.ShapeDtypeStruct((M, N), jnp.float32),
    )(scale, xg, wg)


def kernel(x, w_mat, scale_x, scale_w):
    xg, wg = _allgather(x, w_mat)
    scale = (scale_x * scale_w).reshape(1, 1)
    return _gemm(xg, wg, scale)


# device time: 431352 ns/iter; 1.0059x vs baseline; 1.0059x over previous
import functools

import jax
import jax.numpy as jnp
from jax import lax
from jax.experimental import pallas as pl
from jax.experimental.pallas import tpu as pltpu

N_DEV = 16
M, K, N = 4096, 4096, 8192
K_SH = K // N_DEV
BM, BN = 1024, 2048
N_A = 2048
N_B = N - N_A
SUBS = 2


def _allgather(x, w_mat):

    def body(x_ref, w_ref, xg, wg, xc, wc, xs_sems, xr_sems, ws_sems,
             wr_sems, vs_sems, vr_sems, cp_sem):
        my = lax.axis_index("i")
        left = (my + N_DEV - 1) % N_DEV
        right = (my + 1) % N_DEV

        xc[...] = x_ref[...].astype(jnp.float8_e4m3fn)
        wc[...] = w_ref[...].astype(jnp.float8_e4m3fn)
        cpx = pltpu.make_async_copy(
            xc, xg.at[:, pl.ds(my * K_SH, K_SH)], cp_sem)
        cpx.start()
        cpx.wait()
        cpw = pltpu.make_async_copy(
            wc, wg.at[pl.ds(my * K_SH, K_SH), :], cp_sem)
        cpw.start()
        cpw.wait()

        barrier = pltpu.get_barrier_semaphore()
        for nbr in (left, right):
            pl.semaphore_signal(barrier, inc=1, device_id=(nbr,),
                                device_id_type=pl.DeviceIdType.MESH)
        pl.semaphore_wait(barrier, 2)

        M_H = M // SUBS
        K_H = K_SH // SUBS

        def _desc(stream, h, s, chunk, nbr):
            if stream == 0:
                ref = xg.at[pl.ds(s * M_H, M_H),
                            pl.ds(chunk * K_SH, K_SH)]
                sems = (xs_sems, xr_sems)
            elif stream == 1:
                ref = wg.at[pl.ds(chunk * K_SH + s * K_H, K_H),
                            pl.ds(0, N_A)]
                sems = (ws_sems, wr_sems)
            else:
                ref = wg.at[pl.ds(chunk * K_SH + s * K_H, K_H),
                            pl.ds(N_A, N_B)]
                sems = (vs_sems, vr_sems)
            return pltpu.make_async_remote_copy(
                src_ref=ref, dst_ref=ref,
                send_sem=sems[0].at[h, s], recv_sem=sems[1].at[h, s],
                device_id=(nbr,), device_id_type=pl.DeviceIdType.MESH)

        def _send(stream, h, s):
            chunk = ((my + N_DEV - h) if stream < 2 else (my + h)) % N_DEV
            _desc(stream, h, s, chunk, right if stream < 2 else left).start()

        def _wait_recv(stream, h, s):
            chunk = ((my + 2 * N_DEV - h - 1) if stream < 2
                     else (my + h + 1)) % N_DEV
            _desc(stream, h, s, chunk, left if stream < 2 else right
                  ).wait_recv()

        def _wait_send(stream, h, s):
            chunk = ((my + N_DEV - h) if stream < 2 else (my + h)) % N_DEV
            _desc(stream, h, s, chunk, right if stream < 2 else left
                  ).wait_send()

        for stream in range(3):
            for s in range(SUBS):
                _send(stream, 0, s)

        def hop(h, carry):
            for stream in range(3):
                for s in range(SUBS):
                    _wait_recv(stream, h - 1, s)
                    _send(stream, h, s)
                    _wait_send(stream, h - 1, s)
            return carry

        lax.fori_loop(1, N_DEV - 1, hop, 0)
        for stream in range(3):
            for s in range(SUBS):
                _wait_recv(stream, N_DEV - 2, s)
                _wait_send(stream, N_DEV - 2, s)

        @functools.partial(pl.run_scoped, sem2=pltpu.SemaphoreType.REGULAR)
        def _(sem2):
            for nbr in (left, right):
                pl.semaphore_signal(sem2, inc=1, device_id=(nbr,),
                                    device_id_type=pl.DeviceIdType.MESH)
            pl.semaphore_wait(sem2, 2)

    fp8 = jnp.float8_e4m3fn
    return pl.pallas_call(
        body,
        out_shape=(
            jax.ShapeDtypeStruct((M, K), fp8),
            jax.ShapeDtypeStruct((K, N), fp8),
        ),
        in_specs=[
            pl.BlockSpec(memory_space=pltpu.VMEM),
            pl.BlockSpec(memory_space=pltpu.VMEM),
        ],
        out_specs=(
            pl.BlockSpec(memory_space=pl.ANY),
            pl.BlockSpec(memory_space=pl.ANY),
        ),
        scratch_shapes=[
            pltpu.VMEM((M, K_SH), fp8),
            pltpu.VMEM((K_SH, N), fp8),
            pltpu.SemaphoreType.DMA((N_DEV - 1, SUBS)),
            pltpu.SemaphoreType.DMA((N_DEV - 1, SUBS)),
            pltpu.SemaphoreType.DMA((N_DEV - 1, SUBS)),
            pltpu.SemaphoreType.DMA((N_DEV - 1, SUBS)),
            pltpu.SemaphoreType.DMA((N_DEV - 1, SUBS)),
            pltpu.SemaphoreType.DMA((N_DEV - 1, SUBS)),
            pltpu.SemaphoreType.DMA,
        ],
        compiler_params=pltpu.CompilerParams(collective_id=0),
    )(x, w_mat)


def _gemm(xg, wg, scale):

    def body(s_ref, a_ref, b_ref, o_ref):
        acc = jnp.dot(a_ref[...], b_ref[...],
                      preferred_element_type=jnp.float32)
        o_ref[...] = acc * s_ref[0, 0]

    return pl.pallas_call(
        body,
        grid=(N // BN, M // BM),
        in_specs=[
            pl.BlockSpec(memory_space=pltpu.SMEM),
            pl.BlockSpec((BM, K), lambda j, i: (i, 0)),
            pl.BlockSpec((K, BN), lambda j, i: (0, j)),
        ],
        out_specs=pl.BlockSpec((BM, BN), lambda j, i: (i, j)),
        out_shape=jax.ShapeDtypeStruct((M, N), jnp.float32),
    )(scale, xg, wg)


def kernel(x, w_mat, scale_x, scale_w):
    xg, wg = _allgather(x, w_mat)
    scale = (scale_x * scale_w).reshape(1, 1)
    return _gemm(xg, wg, scale)
